# baseline (speedup 1.0000x reference)
import jax
import jax.numpy as jnp
from jax import lax
from jax.experimental import pallas as pl
from jax.experimental.pallas import tpu as pltpu
from jax.experimental.pallas import tpu_sc as plsc

N = 10000
E = 320000
D = 128
H = 128
ALPHA = 0.01

NPAD = 10240
CHUNK = 128
NCORES = 2
NSUB = 16
NW = NCORES * NSUB
CPW = 80
E_PAD = CPW * NW * CHUNK
SLABS_PER_SUB = NPAD // (CHUNK * NSUB)


def _mm_body(data_ref, noise_ref, w_ref, x_ref):
    feat = data_ref[...] + ALPHA * noise_ref[...]
    x_ref[...] = jnp.dot(feat, w_ref[...], preferred_element_type=jnp.float32)


def _project(data, noise, W):
    blk = 1000
    return pl.pallas_call(
        _mm_body,
        grid=(N // blk,),
        in_specs=[
            pl.BlockSpec((blk, D), lambda i: (i, 0)),
            pl.BlockSpec((blk, D), lambda i: (i, 0)),
            pl.BlockSpec((D, H), lambda i: (0, 0)),
        ],
        out_specs=pl.BlockSpec((blk, H), lambda i: (i, 0)),
        out_shape=jax.ShapeDtypeStruct((N, H), jnp.float32),
    )(data, noise, W)


def _spmm_body(x_hbm, row_hbm, col_hbm, val_hbm, out_hbm,
               col0, col1, row0, row1, val0, val1, buf0, buf1, acc_sh,
               is0, is1, gs0, gs1):
    cid = lax.axis_index("c")
    sid = lax.axis_index("s")
    wid = sid * NCORES + cid

    zeros16 = jnp.zeros((16,), jnp.float32)

    def _zrow(j, _):
        for q in range(H // 16):
            buf0[j, pl.ds(q * 16, 16)] = zeros16
        return 0

    lax.fori_loop(0, CHUNK, _zrow, 0)
    for t in range(SLABS_PER_SUB):
        pltpu.sync_copy(
            buf0, acc_sh.at[pl.ds((t * NSUB + 0) * CHUNK + sid * CHUNK, CHUNK)])
    plsc.subcore_barrier()

    col = (col0, col1)
    row = (row0, row1)
    val = (val0, val1)
    buf = (buf0, buf1)
    isem = (is0, is1)
    gsem = (gs0, gs1)

    def _base(i):
        return (wid + i * NW) * CHUNK

    def _scale(b, vl):
        def grp(g, _):
            vv = vl[pl.ds(g * 16, 16)]
            for lane in range(16):
                v = vv[lane]
                j = g * 16 + lane
                for q in range(H // 16):
                    b[j, pl.ds(q * 16, 16)] = b[j, pl.ds(q * 16, 16)] * v
            return 0

        lax.fori_loop(0, CHUNK // 16, grp, 0)

    def _stage(i, p, copy):
        b = _base(i)
        copy(col_hbm.at[pl.ds(b, CHUNK)], col[p], isem[p])
        copy(row_hbm.at[pl.ds(b, CHUNK)], row[p], isem[p])
        copy(val_hbm.at[pl.ds(b, CHUNK)], val[p], isem[p])

    def _chunk(i, _):
        b = _base(i)
        pltpu.sync_copy(col_hbm.at[pl.ds(b, CHUNK)], col0)
        pltpu.sync_copy(row_hbm.at[pl.ds(b, CHUNK)], row0)
        pltpu.sync_copy(val_hbm.at[pl.ds(b, CHUNK)], val0)
        pltpu.async_copy(x_hbm.at[col0], buf0, gs0).wait()
        _scale(buf0, val0)
        pltpu.sync_copy(buf0, acc_sh.at[row0], add=True)
        return 0

    lax.fori_loop(0, CPW, _chunk, 0)
    plsc.subcore_barrier()

    for t in range(SLABS_PER_SUB):
        slab = (t * NSUB + 0) * CHUNK + sid * CHUNK
        pltpu.sync_copy(acc_sh.at[pl.ds(slab, CHUNK)],
                        out_hbm.at[cid, pl.ds(slab, CHUNK)])


def _spmm_partials(x, row1d, col1d, val1d):
    mesh = plsc.VectorSubcoreMesh(core_axis_name="c", subcore_axis_name="s")
    f = pl.kernel(
        _spmm_body,
        out_type=jax.ShapeDtypeStruct((NCORES, NPAD, H), jnp.float32),
        mesh=mesh,
        scratch_types=[
            pltpu.VMEM((CHUNK,), jnp.int32),
            pltpu.VMEM((CHUNK,), jnp.int32),
            pltpu.VMEM((CHUNK,), jnp.int32),
            pltpu.VMEM((CHUNK,), jnp.int32),
            pltpu.VMEM((CHUNK,), jnp.float32),
            pltpu.VMEM((CHUNK,), jnp.float32),
            pltpu.VMEM((CHUNK, H), jnp.float32),
            pltpu.VMEM((CHUNK, H), jnp.float32),
            pltpu.VMEM_SHARED((NPAD, H), jnp.float32),
            pltpu.SemaphoreType.DMA,
            pltpu.SemaphoreType.DMA,
            pltpu.SemaphoreType.DMA,
            pltpu.SemaphoreType.DMA,
        ],
    )
    return f(x, row1d, col1d, val1d)


def _fin_body(p_ref, out_ref):
    s = p_ref[0] + p_ref[1]
    out_ref[...] = jnp.where(s > 0, s, jnp.exp(s) - 1.0)


def _finish(partials):
    blk = 1000
    return pl.pallas_call(
        _fin_body,
        grid=(N // blk,),
        in_specs=[pl.BlockSpec((NCORES, blk, H), lambda i: (0, i, 0))],
        out_specs=pl.BlockSpec((blk, H), lambda i: (i, 0)),
        out_shape=jax.ShapeDtypeStruct((N, H), jnp.float32),
    )(partials)


def kernel(data, adj_indices, adj_values, W):
    noise = jax.random.normal(jax.random.key(42), data.shape, dtype=data.dtype)
    x = _project(data, noise, W)
    pad = (0, E_PAD - E)
    row1d = jnp.pad(adj_indices[0], pad)
    col1d = jnp.pad(adj_indices[1], pad)
    val1d = jnp.pad(adj_values, pad)
    partials = _spmm_partials(x, row1d, col1d, val1d)
    return _finish(partials)


# speedup vs baseline: 1.6689x; 1.6689x over previous
import functools

import jax
import jax.numpy as jnp
from jax import lax
from jax.experimental import pallas as pl
from jax.experimental.pallas import tpu as pltpu
from jax.experimental.pallas import tpu_sc as plsc

N = 10000
E = 320000
D = 128
H = 128
ALPHA = 0.01

NPAD = 10240
CHUNK = 128
NUM_CHUNKS = E // CHUNK
NCORES = 2
NSUB = 16
NW = NCORES * NSUB
SLABS_PER_SUB = NPAD // (CHUNK * NSUB)


def _mm_body(data_ref, noise_ref, w_ref, x_ref):
    feat = data_ref[...] + ALPHA * noise_ref[...]
    x_ref[...] = jnp.dot(feat, w_ref[...], preferred_element_type=jnp.float32)


def _project(data, noise, W):
    blk = 1000
    return pl.pallas_call(
        _mm_body,
        grid=(N // blk,),
        in_specs=[
            pl.BlockSpec((blk, D), lambda i: (i, 0)),
            pl.BlockSpec((blk, D), lambda i: (i, 0)),
            pl.BlockSpec((D, H), lambda i: (0, 0)),
        ],
        out_specs=pl.BlockSpec((blk, H), lambda i: (i, 0)),
        out_shape=jax.ShapeDtypeStruct((N, H), jnp.float32),
    )(data, noise, W)


def _spmm_body(x_hbm, row_hbm, col_hbm, val_hbm, out_hbm,
               col_v, row_v, val_v, rows_v, acc_sh, sem):
    cid = lax.axis_index("c")
    sid = lax.axis_index("s")
    wid = sid * NCORES + cid

    zeros16 = jnp.zeros((16,), jnp.float32)

    def _zrow(j, _):
        for q in range(H // 16):
            rows_v[j, pl.ds(q * 16, 16)] = zeros16
        return 0

    lax.fori_loop(0, CHUNK, _zrow, 0)
    for t in range(SLABS_PER_SUB):
        slab = (t * NSUB) * CHUNK
        pltpu.sync_copy(rows_v, acc_sh.at[pl.ds(slab + sid * CHUNK, CHUNK)])
    plsc.subcore_barrier()

    n_iters = (NUM_CHUNKS - wid + NW - 1) // NW

    def _chunk(i, _):
        base = (wid + i * NW) * CHUNK
        pltpu.sync_copy(col_hbm.at[pl.ds(base, CHUNK)], col_v)
        pltpu.sync_copy(row_hbm.at[pl.ds(base, CHUNK)], row_v)
        pltpu.sync_copy(val_hbm.at[pl.ds(base, CHUNK)], val_v)
        pltpu.async_copy(x_hbm.at[col_v], rows_v, sem).wait()

        def _scale(g, _):
            vv = val_v[pl.ds(g * 16, 16)]
            for lane in range(16):
                v = vv[lane]
                j = g * 16 + lane
                for q in range(H // 16):
                    rows_v[j, pl.ds(q * 16, 16)] = rows_v[j, pl.ds(q * 16, 16)] * v
            return 0

        lax.fori_loop(0, CHUNK // 16, _scale, 0)
        pltpu.sync_copy(rows_v, acc_sh.at[row_v], add=True)
        return 0

    lax.fori_loop(0, n_iters, _chunk, 0)
    plsc.subcore_barrier()

    for t in range(SLABS_PER_SUB):
        slab = (t * NSUB + 0) * CHUNK + sid * CHUNK
        pltpu.sync_copy(acc_sh.at[pl.ds(slab, CHUNK)],
                        out_hbm.at[cid, pl.ds(slab, CHUNK)])


def _spmm_partials(x, row, col, vals):
    mesh = plsc.VectorSubcoreMesh(core_axis_name="c", subcore_axis_name="s")
    f = pl.kernel(
        _spmm_body,
        out_type=jax.ShapeDtypeStruct((NCORES, NPAD, H), jnp.float32),
        mesh=mesh,
        scratch_types=[
            pltpu.VMEM((CHUNK,), jnp.int32),
            pltpu.VMEM((CHUNK,), jnp.int32),
            pltpu.VMEM((CHUNK,), jnp.float32),
            pltpu.VMEM((CHUNK, H), jnp.float32),
            pltpu.VMEM_SHARED((NPAD, H), jnp.float32),
            pltpu.SemaphoreType.DMA,
        ],
    )
    return f(x, row, col, vals)


def _fin_body(p_ref, out_ref):
    s = p_ref[0] + p_ref[1]
    out_ref[...] = jnp.where(s > 0, s, jnp.exp(s) - 1.0)


def _finish(partials):
    blk = 1000
    return pl.pallas_call(
        _fin_body,
        grid=(N // blk,),
        in_specs=[pl.BlockSpec((NCORES, blk, H), lambda i: (0, i, 0))],
        out_specs=pl.BlockSpec((blk, H), lambda i: (i, 0)),
        out_shape=jax.ShapeDtypeStruct((N, H), jnp.float32),
    )(partials)


def kernel(data, adj_indices, adj_values, W):
    noise = jax.random.normal(jax.random.key(42), data.shape, dtype=data.dtype)
    x = _project(data, noise, W)
    row = adj_indices[0]
    col = adj_indices[1]
    partials = _spmm_partials(x, row, col, adj_values)
    return _finish(partials)


# speedup vs baseline: 2.8010x; 1.6783x over previous
import jax
import jax.numpy as jnp
from jax import lax
from jax.experimental import pallas as pl
from jax.experimental.pallas import tpu as pltpu
from jax.experimental.pallas import tpu_sc as plsc

N = 10000
E = 320000
D = 128
H = 128
ALPHA = 0.01

NPAD = 10240
CHUNK = 128
NCORES = 2
NSUB = 16
NW = NCORES * NSUB
CPW = 80
E_PAD = CPW * NW * CHUNK
SLABS_PER_SUB = NPAD // (CHUNK * NSUB)


def _mm_body(data_ref, noise_ref, w_ref, x_ref):
    feat = data_ref[...] + ALPHA * noise_ref[...]
    x_ref[...] = jnp.dot(feat, w_ref[...], preferred_element_type=jnp.float32)


def _project(data, noise, W):
    blk = 1000
    return pl.pallas_call(
        _mm_body,
        grid=(N // blk,),
        in_specs=[
            pl.BlockSpec((blk, D), lambda i: (i, 0)),
            pl.BlockSpec((blk, D), lambda i: (i, 0)),
            pl.BlockSpec((D, H), lambda i: (0, 0)),
        ],
        out_specs=pl.BlockSpec((blk, H), lambda i: (i, 0)),
        out_shape=jax.ShapeDtypeStruct((N, H), jnp.float32),
    )(data, noise, W)


def _spmm_body(x_hbm, row_hbm, col_hbm, val_hbm, out_hbm,
               col0, col1, row0, row1, val0, val1, buf0, buf1, acc_sh,
               is0, is1, gs0, gs1):
    cid = lax.axis_index("c")
    sid = lax.axis_index("s")
    wid = sid * NCORES + cid

    zeros16 = jnp.zeros((16,), jnp.float32)

    def _zrow(j, _):
        for q in range(H // 16):
            buf0[j, pl.ds(q * 16, 16)] = zeros16
        return 0

    lax.fori_loop(0, CHUNK, _zrow, 0)
    for t in range(SLABS_PER_SUB):
        pltpu.sync_copy(
            buf0, acc_sh.at[pl.ds((t * NSUB + 0) * CHUNK + sid * CHUNK, CHUNK)])
    plsc.subcore_barrier()

    col = (col0, col1)
    row = (row0, row1)
    val = (val0, val1)
    buf = (buf0, buf1)
    isem = (is0, is1)
    gsem = (gs0, gs1)

    def _base(i):
        return (wid + i * NW) * CHUNK

    def _scale(b, vl):
        def grp(g, _):
            vv = vl[pl.ds(g * 16, 16)]
            for lane in range(16):
                v = vv[lane]
                j = g * 16 + lane
                for q in range(H // 16):
                    b[j, pl.ds(q * 16, 16)] = b[j, pl.ds(q * 16, 16)] * v
            return 0

        lax.fori_loop(0, CHUNK // 16, grp, 0)

    def _stage(i, p, copy):
        b = _base(i)
        copy(col_hbm.at[pl.ds(b, CHUNK)], col[p], isem[p])
        copy(row_hbm.at[pl.ds(b, CHUNK)], row[p], isem[p])
        copy(val_hbm.at[pl.ds(b, CHUNK)], val[p], isem[p])

    def _wait_stage(i, p):
        b = _base(i)
        pltpu.make_async_copy(col_hbm.at[pl.ds(b, CHUNK)], col[p],
                              isem[p]).wait()
        pltpu.make_async_copy(row_hbm.at[pl.ds(b, CHUNK)], row[p],
                              isem[p]).wait()
        pltpu.make_async_copy(val_hbm.at[pl.ds(b, CHUNK)], val[p],
                              isem[p]).wait()

    b0 = _base(0)
    pltpu.sync_copy(col_hbm.at[pl.ds(b0, CHUNK)], col0)
    pltpu.sync_copy(row_hbm.at[pl.ds(b0, CHUNK)], row0)
    pltpu.sync_copy(val_hbm.at[pl.ds(b0, CHUNK)], val0)
    pltpu.async_copy(x_hbm.at[col0], buf0, gs0)
    _stage(1, 1, pltpu.async_copy)

    def _pair(t, _):
        i0 = 2 * t
        for p in range(2):
            i = i0 + p
            q = 1 - p

            @pl.when(i + 1 < CPW)
            def _():
                _wait_stage(i + 1, q)
                pltpu.async_copy(x_hbm.at[col[q]], buf[q], gsem[q])

            pltpu.make_async_copy(x_hbm.at[col[p]], buf[p], gsem[p]).wait()
            _scale(buf[p], val[p])
            pltpu.sync_copy(buf[p], acc_sh.at[row[p]], add=True)

            @pl.when(i + 2 < CPW)
            def _():
                _stage(i + 2, p, pltpu.async_copy)
        return 0

    lax.fori_loop(0, CPW // 2, _pair, 0)
    plsc.subcore_barrier()

    for t in range(SLABS_PER_SUB):
        slab = (t * NSUB + 0) * CHUNK + sid * CHUNK
        pltpu.sync_copy(acc_sh.at[pl.ds(slab, CHUNK)],
                        out_hbm.at[cid, pl.ds(slab, CHUNK)])


def _spmm_partials(x, row1d, col1d, val1d):
    mesh = plsc.VectorSubcoreMesh(core_axis_name="c", subcore_axis_name="s")
    f = pl.kernel(
        _spmm_body,
        out_type=jax.ShapeDtypeStruct((NCORES, NPAD, H), jnp.float32),
        mesh=mesh,
        scratch_types=[
            pltpu.VMEM((CHUNK,), jnp.int32),
            pltpu.VMEM((CHUNK,), jnp.int32),
            pltpu.VMEM((CHUNK,), jnp.int32),
            pltpu.VMEM((CHUNK,), jnp.int32),
            pltpu.VMEM((CHUNK,), jnp.float32),
            pltpu.VMEM((CHUNK,), jnp.float32),
            pltpu.VMEM((CHUNK, H), jnp.float32),
            pltpu.VMEM((CHUNK, H), jnp.float32),
            pltpu.VMEM_SHARED((NPAD, H), jnp.float32),
            pltpu.SemaphoreType.DMA,
            pltpu.SemaphoreType.DMA,
            pltpu.SemaphoreType.DMA,
            pltpu.SemaphoreType.DMA,
        ],
    )
    return f(x, row1d, col1d, val1d)


def _fin_body(p_ref, out_ref):
    s = p_ref[0] + p_ref[1]
    out_ref[...] = jnp.where(s > 0, s, jnp.exp(s) - 1.0)


def _finish(partials):
    blk = 1000
    return pl.pallas_call(
        _fin_body,
        grid=(N // blk,),
        in_specs=[pl.BlockSpec((NCORES, blk, H), lambda i: (0, i, 0))],
        out_specs=pl.BlockSpec((blk, H), lambda i: (i, 0)),
        out_shape=jax.ShapeDtypeStruct((N, H), jnp.float32),
    )(partials)


def kernel(data, adj_indices, adj_values, W):
    noise = jax.random.normal(jax.random.key(42), data.shape, dtype=data.dtype)
    x = _project(data, noise, W)
    npad = E_PAD - E
    pad_row = N + (jnp.arange(npad, dtype=jnp.int32) % (NPAD - N))
    pad_col = jnp.arange(npad, dtype=jnp.int32) % N
    row1d = jnp.concatenate([adj_indices[0], pad_row])
    col1d = jnp.concatenate([adj_indices[1], pad_col])
    val1d = jnp.pad(adj_values, (0, npad))
    partials = _spmm_partials(x, row1d, col1d, val1d)
    return _finish(partials)


# speedup vs baseline: 3.1919x; 1.1396x over previous
import jax
import jax.numpy as jnp
from jax import lax
from jax.experimental import pallas as pl
from jax.experimental.pallas import tpu as pltpu
from jax.experimental.pallas import tpu_sc as plsc

N = 10000
E = 320000
D = 128
H = 128
ALPHA = 0.01

NPAD = 10240
CHUNK = 128
NCORES = 2
NSUB = 16
NW = NCORES * NSUB
CPW = 80
E_PAD = CPW * NW * CHUNK
SLABS_PER_SUB = NPAD // (CHUNK * NSUB)


def _mm_body(data_ref, noise_ref, w_ref, x_ref):
    feat = data_ref[...] + ALPHA * noise_ref[...]
    x_ref[...] = jnp.dot(feat, w_ref[...], preferred_element_type=jnp.float32)


def _project(data, noise, W):
    blk = 1000
    return pl.pallas_call(
        _mm_body,
        grid=(N // blk,),
        in_specs=[
            pl.BlockSpec((blk, D), lambda i: (i, 0)),
            pl.BlockSpec((blk, D), lambda i: (i, 0)),
            pl.BlockSpec((D, H), lambda i: (0, 0)),
        ],
        out_specs=pl.BlockSpec((blk, H), lambda i: (i, 0)),
        out_shape=jax.ShapeDtypeStruct((N, H), jnp.float32),
    )(data, noise, W)


def _spmm_body(x_hbm, row_hbm, col_hbm, val_hbm, out_hbm,
               col0, col1, row0, row1, val0, val1, buf0, buf1,
               srow0, srow1, acc_sh, is0, is1, gs0, gs1, ss0, ss1):
    cid = lax.axis_index("c")
    sid = lax.axis_index("s")
    wid = sid * NCORES + cid

    zeros16 = jnp.zeros((16,), jnp.float32)

    def _zrow(j, _):
        for q in range(H // 16):
            buf0[j, pl.ds(q * 16, 16)] = zeros16
        return 0

    lax.fori_loop(0, CHUNK, _zrow, 0)
    for t in range(SLABS_PER_SUB):
        pltpu.sync_copy(
            buf0, acc_sh.at[pl.ds((t * NSUB + 0) * CHUNK + sid * CHUNK, CHUNK)])
    plsc.subcore_barrier()

    col = (col0, col1)
    row = (row0, row1)
    val = (val0, val1)
    buf = (buf0, buf1)
    srow = (srow0, srow1)
    isem = (is0, is1)
    gsem = (gs0, gs1)
    ssem = (ss0, ss1)

    def _base(i):
        return (wid + i * NW) * CHUNK

    def _scale(b, vl):
        def grp(g, _):
            vv = vl[pl.ds(g * 16, 16)]
            for lane in range(16):
                v = vv[lane]
                j = g * 16 + lane
                for q in range(H // 16):
                    b[j, pl.ds(q * 16, 16)] = b[j, pl.ds(q * 16, 16)] * v
            return 0

        lax.fori_loop(0, CHUNK // 16, grp, 0)

    def _stage(i, p, copy):
        b = _base(i)
        copy(col_hbm.at[pl.ds(b, CHUNK)], col[p], isem[p])
        copy(row_hbm.at[pl.ds(b, CHUNK)], row[p], isem[p])
        copy(val_hbm.at[pl.ds(b, CHUNK)], val[p], isem[p])

    def _wait_stage(i, p):
        b = _base(i)
        pltpu.make_async_copy(col_hbm.at[pl.ds(b, CHUNK)], col[p],
                              isem[p]).wait()
        pltpu.make_async_copy(row_hbm.at[pl.ds(b, CHUNK)], row[p],
                              isem[p]).wait()
        pltpu.make_async_copy(val_hbm.at[pl.ds(b, CHUNK)], val[p],
                              isem[p]).wait()

    b0 = _base(0)
    pltpu.sync_copy(col_hbm.at[pl.ds(b0, CHUNK)], col0)
    pltpu.sync_copy(row_hbm.at[pl.ds(b0, CHUNK)], row0)
    pltpu.sync_copy(val_hbm.at[pl.ds(b0, CHUNK)], val0)
    pltpu.async_copy(x_hbm.at[col0], buf0, gs0)
    _stage(1, 1, pltpu.async_copy)

    def _pair(t, _):
        i0 = 2 * t
        for p in range(2):
            i = i0 + p
            q = 1 - p

            @pl.when(i + 1 < CPW)
            def _():
                @pl.when(i >= 1)
                def _():
                    pltpu.make_async_copy(buf[q], acc_sh.at[srow[q]],
                                          ssem[q]).wait()
                _wait_stage(i + 1, q)
                pltpu.async_copy(x_hbm.at[col[q]], buf[q], gsem[q])

            pltpu.make_async_copy(x_hbm.at[col[p]], buf[p], gsem[p]).wait()
            _scale(buf[p], val[p])
            for k in range(CHUNK // 16):
                srow[p][pl.ds(k * 16, 16)] = row[p][pl.ds(k * 16, 16)]
            pltpu.async_copy(buf[p], acc_sh.at[srow[p]], ssem[p], add=True)

            @pl.when(i + 2 < CPW)
            def _():
                _stage(i + 2, p, pltpu.async_copy)
        return 0

    lax.fori_loop(0, CPW // 2, _pair, 0)
    pltpu.make_async_copy(buf[0], acc_sh.at[srow[0]], ssem[0]).wait()
    pltpu.make_async_copy(buf[1], acc_sh.at[srow[1]], ssem[1]).wait()
    plsc.subcore_barrier()

    for t in range(SLABS_PER_SUB):
        slab = (t * NSUB + 0) * CHUNK + sid * CHUNK
        pltpu.sync_copy(acc_sh.at[pl.ds(slab, CHUNK)],
                        out_hbm.at[cid, pl.ds(slab, CHUNK)])


def _spmm_partials(x, row1d, col1d, val1d):
    mesh = plsc.VectorSubcoreMesh(core_axis_name="c", subcore_axis_name="s")
    f = pl.kernel(
        _spmm_body,
        out_type=jax.ShapeDtypeStruct((NCORES, NPAD, H), jnp.float32),
        mesh=mesh,
        scratch_types=[
            pltpu.VMEM((CHUNK,), jnp.int32),
            pltpu.VMEM((CHUNK,), jnp.int32),
            pltpu.VMEM((CHUNK,), jnp.int32),
            pltpu.VMEM((CHUNK,), jnp.int32),
            pltpu.VMEM((CHUNK,), jnp.float32),
            pltpu.VMEM((CHUNK,), jnp.float32),
            pltpu.VMEM((CHUNK, H), jnp.float32),
            pltpu.VMEM((CHUNK, H), jnp.float32),
            pltpu.VMEM((CHUNK,), jnp.int32),
            pltpu.VMEM((CHUNK,), jnp.int32),
            pltpu.VMEM_SHARED((NPAD, H), jnp.float32),
            pltpu.SemaphoreType.DMA,
            pltpu.SemaphoreType.DMA,
            pltpu.SemaphoreType.DMA,
            pltpu.SemaphoreType.DMA,
            pltpu.SemaphoreType.DMA,
            pltpu.SemaphoreType.DMA,
        ],
    )
    return f(x, row1d, col1d, val1d)


def _fin_body(p_ref, out_ref):
    s = p_ref[0] + p_ref[1]
    out_ref[...] = jnp.where(s > 0, s, jnp.exp(s) - 1.0)


def _finish(partials):
    blk = 1000
    return pl.pallas_call(
        _fin_body,
        grid=(N // blk,),
        in_specs=[pl.BlockSpec((NCORES, blk, H), lambda i: (0, i, 0))],
        out_specs=pl.BlockSpec((blk, H), lambda i: (i, 0)),
        out_shape=jax.ShapeDtypeStruct((N, H), jnp.float32),
    )(partials)


def kernel(data, adj_indices, adj_values, W):
    noise = jax.random.normal(jax.random.key(42), data.shape, dtype=data.dtype)
    x = _project(data, noise, W)
    npad = E_PAD - E
    pad_row = N + (jnp.arange(npad, dtype=jnp.int32) % (NPAD - N))
    pad_col = jnp.arange(npad, dtype=jnp.int32) % N
    row1d = jnp.concatenate([adj_indices[0], pad_row])
    col1d = jnp.concatenate([adj_indices[1], pad_col])
    val1d = jnp.pad(adj_values, (0, npad))
    partials = _spmm_partials(x, row1d, col1d, val1d)
    return _finish(partials)
